# Initial kernel scaffold; baseline (speedup 1.0000x reference)
import jax
import jax.numpy as jnp
from jax.experimental import pallas as pl


def kernel(x, gamma1, beta1, Wq, bq, Wk, bk, Wv, bv, Wo, bo, gamma2, beta2, Wg, bg, W1, b1, W2, b2):
    raise NotImplementedError("write your pallas kernel here")


# speedup vs baseline: 1.2892x; 1.2892x over previous
import jax
import jax.numpy as jnp
from jax.experimental import pallas as pl
from jax.experimental.pallas import tpu as pltpu

S, D, H, E, F = 2048, 768, 12, 8, 3072
DH = D // H
LN_EPS = 1e-5
NEG = -1e30

SB_QKV = 512
QB = 512
SB_RT = 512
SB_MOE = 1024


def _layernorm(x, g, b):
    m = jnp.mean(x, -1, keepdims=True)
    v = jnp.mean(jnp.square(x - m), -1, keepdims=True)
    return (x - m) * jax.lax.rsqrt(v + LN_EPS) * g + b


def _qkv_body(x_ref, g_ref, b_ref, w_ref, bias_ref, o_ref):
    h = _layernorm(x_ref[...], g_ref[...], b_ref[...])
    o = jnp.dot(h.astype(jnp.bfloat16), w_ref[...],
                preferred_element_type=jnp.float32)
    o_ref[...] = (o + bias_ref[...]).astype(jnp.bfloat16)


def _attn_body(q_ref, k_ref, v_ref, o_ref):
    for hh in range(2):
        sl = slice(hh * DH, (hh + 1) * DH)
        s = jax.lax.dot_general(q_ref[:, sl], k_ref[:, sl],
                                (((1,), (1,)), ((), ())),
                                preferred_element_type=jnp.float32)
        s = s * 0.125
        s = s - jnp.max(s, -1, keepdims=True)
        p = jnp.exp(s)
        p = p * (1.0 / jnp.sum(p, -1, keepdims=True))
        o = jnp.dot(p.astype(jnp.bfloat16), v_ref[:, sl],
                    preferred_element_type=jnp.float32)
        o_ref[:, sl] = o.astype(jnp.bfloat16)


def _router_body(x_ref, a_ref, wo_ref, bo_ref, g_ref, b_ref, wg_ref, bg_ref,
                 x2_ref, h2_ref, tw_ref):
    ao = jnp.dot(a_ref[...], wo_ref[...],
                 preferred_element_type=jnp.float32) + bo_ref[...]
    x2 = x_ref[...] + ao
    x2_ref[...] = x2
    h2 = _layernorm(x2, g_ref[...], b_ref[...])
    h2_ref[...] = h2.astype(jnp.bfloat16)
    logits = jnp.dot(h2, wg_ref[...],
                     preferred_element_type=jnp.float32) + bg_ref[...]
    lane = jax.lax.broadcasted_iota(jnp.int32, logits.shape, 1)
    m1 = jnp.max(logits, -1, keepdims=True)
    i1 = jnp.min(jnp.where(logits == m1, lane, 128), -1, keepdims=True)
    l2 = jnp.where(lane == i1, NEG, logits)
    m2 = jnp.max(l2, -1, keepdims=True)
    i2 = jnp.min(jnp.where(l2 == m2, lane, 128), -1, keepdims=True)
    w2 = 1.0 / (1.0 + jnp.exp(m1 - m2))
    w1 = 1.0 - w2
    tw_ref[...] = (jnp.where(lane == i1, w1, 0.0)
                   + jnp.where(lane == i2, w2, 0.0))


def _moe_body(h2_ref, tw_ref, x2_ref, w1_ref, b1_ref, w2_ref, b2_ref, o_ref):
    e = pl.program_id(1)

    @pl.when(e == 0)
    def _():
        o_ref[...] = x2_ref[...]

    t = jnp.dot(h2_ref[...], w1_ref[0],
                preferred_element_type=jnp.float32) + b1_ref[0]
    t = jax.nn.gelu(t).astype(jnp.bfloat16)
    y = jnp.dot(t, w2_ref[0], preferred_element_type=jnp.float32) + b2_ref[0]
    lane = jax.lax.broadcasted_iota(jnp.int32, tw_ref.shape, 1)
    wsel = jnp.sum(jnp.where(lane == e, tw_ref[...], 0.0), -1, keepdims=True)
    o_ref[...] += y * wsel


def kernel(x, gamma1, beta1, Wq, bq, Wk, bk, Wv, bv, Wo, bo,
           gamma2, beta2, Wg, bg, W1, b1, W2, b2):
    xs = x.reshape(S, D)
    wqkv = jnp.concatenate([Wq, Wk, Wv], axis=1).astype(jnp.bfloat16)
    bqkv = jnp.concatenate([bq, bk, bv]).reshape(1, 3 * D)
    g1 = gamma1.reshape(1, D)
    be1 = beta1.reshape(1, D)

    qkv = pl.pallas_call(
        _qkv_body,
        grid=(S // SB_QKV,),
        in_specs=[
            pl.BlockSpec((SB_QKV, D), lambda i: (i, 0)),
            pl.BlockSpec((1, D), lambda i: (0, 0)),
            pl.BlockSpec((1, D), lambda i: (0, 0)),
            pl.BlockSpec((D, 3 * D), lambda i: (0, 0)),
            pl.BlockSpec((1, 3 * D), lambda i: (0, 0)),
        ],
        out_specs=pl.BlockSpec((SB_QKV, 3 * D), lambda i: (i, 0)),
        out_shape=jax.ShapeDtypeStruct((S, 3 * D), jnp.bfloat16),
    )(xs, g1, be1, wqkv, bqkv)

    nhb = D // 128
    attn = pl.pallas_call(
        _attn_body,
        grid=(H // 2, S // QB),
        in_specs=[
            pl.BlockSpec((QB, 128), lambda g, i: (i, g)),
            pl.BlockSpec((S, 128), lambda g, i: (0, nhb + g)),
            pl.BlockSpec((S, 128), lambda g, i: (0, 2 * nhb + g)),
        ],
        out_specs=pl.BlockSpec((QB, 128), lambda g, i: (i, g)),
        out_shape=jax.ShapeDtypeStruct((S, D), jnp.bfloat16),
    )(qkv, qkv, qkv)

    wgp = jnp.zeros((D, 128), jnp.float32).at[:, :E].set(Wg)
    bgp = jnp.full((1, 128), NEG, jnp.float32).at[0, :E].set(bg)
    x2, h2, tw = pl.pallas_call(
        _router_body,
        grid=(S // SB_RT,),
        in_specs=[
            pl.BlockSpec((SB_RT, D), lambda i: (i, 0)),
            pl.BlockSpec((SB_RT, D), lambda i: (i, 0)),
            pl.BlockSpec((D, D), lambda i: (0, 0)),
            pl.BlockSpec((1, D), lambda i: (0, 0)),
            pl.BlockSpec((1, D), lambda i: (0, 0)),
            pl.BlockSpec((1, D), lambda i: (0, 0)),
            pl.BlockSpec((D, 128), lambda i: (0, 0)),
            pl.BlockSpec((1, 128), lambda i: (0, 0)),
        ],
        out_specs=[
            pl.BlockSpec((SB_RT, D), lambda i: (i, 0)),
            pl.BlockSpec((SB_RT, D), lambda i: (i, 0)),
            pl.BlockSpec((SB_RT, 128), lambda i: (i, 0)),
        ],
        out_shape=[
            jax.ShapeDtypeStruct((S, D), jnp.float32),
            jax.ShapeDtypeStruct((S, D), jnp.bfloat16),
            jax.ShapeDtypeStruct((S, 128), jnp.float32),
        ],
    )(xs, attn, Wo.astype(jnp.bfloat16), bo.reshape(1, D),
      gamma2.reshape(1, D), beta2.reshape(1, D), wgp, bgp)

    out = pl.pallas_call(
        _moe_body,
        grid=(S // SB_MOE, E),
        in_specs=[
            pl.BlockSpec((SB_MOE, D), lambda i, e: (i, 0)),
            pl.BlockSpec((SB_MOE, 128), lambda i, e: (i, 0)),
            pl.BlockSpec((SB_MOE, D), lambda i, e: (i, 0)),
            pl.BlockSpec((1, D, F), lambda i, e: (e, 0, 0)),
            pl.BlockSpec((1, 1, F), lambda i, e: (e, 0, 0)),
            pl.BlockSpec((1, F, D), lambda i, e: (e, 0, 0)),
            pl.BlockSpec((1, 1, D), lambda i, e: (e, 0, 0)),
        ],
        out_specs=pl.BlockSpec((SB_MOE, D), lambda i, e: (i, 0)),
        out_shape=jax.ShapeDtypeStruct((S, D), jnp.float32),
    )(h2, tw, x2, W1.astype(jnp.bfloat16), b1.reshape(E, 1, F),
      W2.astype(jnp.bfloat16), b2.reshape(E, 1, D))

    return out.reshape(1, S, D)


# speedup vs baseline: 1.3179x; 1.0223x over previous
import jax
import jax.numpy as jnp
from jax.experimental import pallas as pl
from jax.experimental.pallas import tpu as pltpu

S, D, H, E, F = 2048, 768, 12, 8, 3072
DH = D // H
LN_EPS = 1e-5
NEG = -1e30

SB_QKV = 512
QB = 512
SB_RT = 512
SB_MOE = 1024


def _layernorm(x, g, b):
    m = jnp.mean(x, -1, keepdims=True)
    v = jnp.mean(jnp.square(x - m), -1, keepdims=True)
    return (x - m) * jax.lax.rsqrt(v + LN_EPS) * g + b


def _qkv_body(x_ref, g_ref, b_ref, w_ref, bias_ref, o_ref):
    h = _layernorm(x_ref[...], g_ref[...], b_ref[...])
    o = jnp.dot(h.astype(jnp.bfloat16), w_ref[...],
                preferred_element_type=jnp.float32)
    o_ref[...] = (o + bias_ref[...]).astype(jnp.bfloat16)


def _attn_body(q_ref, k_ref, v_ref, o_ref):
    for hh in range(2):
        sl = slice(hh * DH, (hh + 1) * DH)
        s = jax.lax.dot_general(q_ref[:, sl], k_ref[:, sl],
                                (((1,), (1,)), ((), ())),
                                preferred_element_type=jnp.float32)
        s = s * 0.125
        s = s - jnp.max(s, -1, keepdims=True)
        p = jnp.exp(s)
        r = (1.0 / jnp.sum(p, -1, keepdims=True)).astype(jnp.bfloat16)
        o = jnp.dot(p.astype(jnp.bfloat16) * r, v_ref[:, sl],
                    preferred_element_type=jnp.float32)
        o_ref[:, sl] = o.astype(jnp.bfloat16)


def _router_body(x_ref, a_ref, wo_ref, bo_ref, g_ref, b_ref, wg_ref, bg_ref,
                 x2_ref, h2_ref, tw_ref):
    ao = jnp.dot(a_ref[...], wo_ref[...],
                 preferred_element_type=jnp.float32) + bo_ref[...]
    x2 = x_ref[...] + ao
    x2_ref[...] = x2
    h2 = _layernorm(x2, g_ref[...], b_ref[...])
    h2_ref[...] = h2.astype(jnp.bfloat16)
    logits = jnp.dot(h2, wg_ref[...],
                     preferred_element_type=jnp.float32) + bg_ref[...]
    lane = jax.lax.broadcasted_iota(jnp.int32, logits.shape, 1)
    m1 = jnp.max(logits, -1, keepdims=True)
    i1 = jnp.min(jnp.where(logits == m1, lane, 128), -1, keepdims=True)
    l2 = jnp.where(lane == i1, NEG, logits)
    m2 = jnp.max(l2, -1, keepdims=True)
    i2 = jnp.min(jnp.where(l2 == m2, lane, 128), -1, keepdims=True)
    w2 = 1.0 / (1.0 + jnp.exp(m1 - m2))
    w1 = 1.0 - w2
    tw_ref[...] = (jnp.where(lane == i1, w1, 0.0)
                   + jnp.where(lane == i2, w2, 0.0))


def _moe_body(h2_ref, tw_ref, x2_ref, w1_ref, b1_ref, w2_ref, b2_ref, o_ref):
    e = pl.program_id(1)

    @pl.when(e == 0)
    def _():
        o_ref[...] = x2_ref[...]

    t = jnp.dot(h2_ref[...], w1_ref[0],
                preferred_element_type=jnp.float32) + b1_ref[0]
    t = jax.nn.gelu(t.astype(jnp.bfloat16))
    y = jnp.dot(t, w2_ref[0], preferred_element_type=jnp.float32) + b2_ref[0]
    lane = jax.lax.broadcasted_iota(jnp.int32, tw_ref.shape, 1)
    wsel = jnp.sum(jnp.where(lane == e, tw_ref[...], 0.0), -1, keepdims=True)
    o_ref[...] += y * wsel


def kernel(x, gamma1, beta1, Wq, bq, Wk, bk, Wv, bv, Wo, bo,
           gamma2, beta2, Wg, bg, W1, b1, W2, b2):
    xs = x.reshape(S, D)
    wqkv = jnp.concatenate([Wq, Wk, Wv], axis=1).astype(jnp.bfloat16)
    bqkv = jnp.concatenate([bq, bk, bv]).reshape(1, 3 * D)
    g1 = gamma1.reshape(1, D)
    be1 = beta1.reshape(1, D)

    qkv = pl.pallas_call(
        _qkv_body,
        grid=(S // SB_QKV,),
        in_specs=[
            pl.BlockSpec((SB_QKV, D), lambda i: (i, 0)),
            pl.BlockSpec((1, D), lambda i: (0, 0)),
            pl.BlockSpec((1, D), lambda i: (0, 0)),
            pl.BlockSpec((D, 3 * D), lambda i: (0, 0)),
            pl.BlockSpec((1, 3 * D), lambda i: (0, 0)),
        ],
        out_specs=pl.BlockSpec((SB_QKV, 3 * D), lambda i: (i, 0)),
        out_shape=jax.ShapeDtypeStruct((S, 3 * D), jnp.bfloat16),
    )(xs, g1, be1, wqkv, bqkv)

    nhb = D // 128
    attn = pl.pallas_call(
        _attn_body,
        grid=(H // 2, S // QB),
        in_specs=[
            pl.BlockSpec((QB, 128), lambda g, i: (i, g)),
            pl.BlockSpec((S, 128), lambda g, i: (0, nhb + g)),
            pl.BlockSpec((S, 128), lambda g, i: (0, 2 * nhb + g)),
        ],
        out_specs=pl.BlockSpec((QB, 128), lambda g, i: (i, g)),
        out_shape=jax.ShapeDtypeStruct((S, D), jnp.bfloat16),
    )(qkv, qkv, qkv)

    wgp = jnp.zeros((D, 128), jnp.float32).at[:, :E].set(Wg)
    bgp = jnp.full((1, 128), NEG, jnp.float32).at[0, :E].set(bg)
    x2, h2, tw = pl.pallas_call(
        _router_body,
        grid=(S // SB_RT,),
        in_specs=[
            pl.BlockSpec((SB_RT, D), lambda i: (i, 0)),
            pl.BlockSpec((SB_RT, D), lambda i: (i, 0)),
            pl.BlockSpec((D, D), lambda i: (0, 0)),
            pl.BlockSpec((1, D), lambda i: (0, 0)),
            pl.BlockSpec((1, D), lambda i: (0, 0)),
            pl.BlockSpec((1, D), lambda i: (0, 0)),
            pl.BlockSpec((D, 128), lambda i: (0, 0)),
            pl.BlockSpec((1, 128), lambda i: (0, 0)),
        ],
        out_specs=[
            pl.BlockSpec((SB_RT, D), lambda i: (i, 0)),
            pl.BlockSpec((SB_RT, D), lambda i: (i, 0)),
            pl.BlockSpec((SB_RT, 128), lambda i: (i, 0)),
        ],
        out_shape=[
            jax.ShapeDtypeStruct((S, D), jnp.float32),
            jax.ShapeDtypeStruct((S, D), jnp.bfloat16),
            jax.ShapeDtypeStruct((S, 128), jnp.float32),
        ],
    )(xs, attn, Wo.astype(jnp.bfloat16), bo.reshape(1, D),
      gamma2.reshape(1, D), beta2.reshape(1, D), wgp, bgp)

    out = pl.pallas_call(
        _moe_body,
        grid=(S // SB_MOE, E),
        in_specs=[
            pl.BlockSpec((SB_MOE, D), lambda i, e: (i, 0)),
            pl.BlockSpec((SB_MOE, 128), lambda i, e: (i, 0)),
            pl.BlockSpec((SB_MOE, D), lambda i, e: (i, 0)),
            pl.BlockSpec((1, D, F), lambda i, e: (e, 0, 0)),
            pl.BlockSpec((1, 1, F), lambda i, e: (e, 0, 0)),
            pl.BlockSpec((1, F, D), lambda i, e: (e, 0, 0)),
            pl.BlockSpec((1, 1, D), lambda i, e: (e, 0, 0)),
        ],
        out_specs=pl.BlockSpec((SB_MOE, D), lambda i, e: (i, 0)),
        out_shape=jax.ShapeDtypeStruct((S, D), jnp.float32),
    )(h2, tw, x2, W1.astype(jnp.bfloat16), b1.reshape(E, 1, F),
      W2.astype(jnp.bfloat16), b2.reshape(E, 1, D))

    return out.reshape(1, S, D)
